# baseline (speedup 1.0000x reference)
import functools

import jax
import jax.numpy as jnp
from jax import lax
from jax.experimental import pallas as pl
from jax.experimental.pallas import tpu as pltpu
from jax.experimental.pallas import tpu_sc as plsc

NC = 2
NS = 16
NW = NC * NS
CB = 80

_LOG1P_C = (
    3.3869654e-08, 9.9999428e-01, -4.9983856e-01, 3.3154863e-01,
    -2.3982616e-01, 1.6582276e-01, -9.3252040e-02, 3.4849711e-02,
    -6.1514708e-03,
)


def _log1p_poly(t):
    y = jnp.full_like(t, _LOG1P_C[-1])
    for c in _LOG1P_C[-2::-1]:
        y = y * t + c
    return y


def _proj_body(node_ref, ws_ref, bs_ref, wd_ref, bd_ref, hs_ref, hd_ref):
    x = node_ref[...]
    hs_ref[...] = jnp.dot(x, ws_ref[...], preferred_element_type=jnp.float32) + bs_ref[...]
    hd_ref[...] = jnp.dot(x, wd_ref[...], preferred_element_type=jnp.float32) + bd_ref[...]


def _eproj_body(ef_ref, we_ref, be_ref, out_ref):
    out_ref[...] = (
        jnp.dot(ef_ref[...], we_ref[...], preferred_element_type=jnp.float32)
        + be_ref[...]
    )


def _s2_body(epw, nchunk, hsrc, hdst, eproj, src, dst,
             m_out, stats_out, sidx_v, didx_v, a_v, b_v, c_v, acc_v, sem1, sem2):
    cid = lax.axis_index("c")
    sid = lax.axis_index("s")
    wid = sid * NC + cid

    for r in range(32):
        acc_v[r] = jnp.zeros((16,), jnp.float32)

    def chunk_body(k, carry):
        base = wid * epw + k * CB
        pltpu.sync_copy(src.at[pl.ds(base, CB)], sidx_v)
        pltpu.sync_copy(dst.at[pl.ds(base, CB)], didx_v)
        cp1 = pltpu.async_copy(hsrc.at[sidx_v], a_v, sem1)
        cp2 = pltpu.async_copy(hdst.at[didx_v], b_v, sem2)
        pltpu.sync_copy(eproj.at[pl.ds(base, CB)], c_v)
        cp1.wait()
        cp2.wait()

        def edge_body(e, c2):
            for cg in range(16):
                sl = pl.ds(cg * 16, 16)
                m = a_v[e, sl] + b_v[e, sl] + c_v[e, sl]
                c_v[e, sl] = m
                acc_v[cg] = acc_v[cg] + m
                acc_v[16 + cg] = acc_v[16 + cg] + m * m
            return c2

        lax.fori_loop(0, CB, edge_body, 0, unroll=False)
        pltpu.sync_copy(c_v, m_out.at[pl.ds(base, CB)])
        return carry

    lax.fori_loop(0, nchunk, chunk_body, 0, unroll=False)
    pltpu.sync_copy(acc_v, stats_out.at[wid])


def _s3_body(n_pad, epw, nchunk, m_in, dst, params, out,
             didx_v, m_v, act_v, par_v, zero_v, h_sh, sem1):
    cid = lax.axis_index("c")
    sid = lax.axis_index("s")
    wid = sid * NC + cid
    rows_per_tile = n_pad // NS
    zr = rows_per_tile // 5

    if True:
        pltpu.sync_copy(params, par_v)

        def zrow(r, c2):
            for g in range(8):
                zero_v[r, pl.ds(g * 16, 16)] = jnp.zeros((16,), jnp.float32)
            return c2

        lax.fori_loop(0, zr, zrow, 0, unroll=False)
        for j in range(5):
            pltpu.sync_copy(zero_v, h_sh.at[pl.ds(sid * rows_per_tile + j * zr, zr)])
        plsc.subcore_barrier()

        def chunk_body(k, carry):
            base = wid * epw + k * CB
            pltpu.sync_copy(dst.at[pl.ds(base, CB)], didx_v)
            pltpu.sync_copy(m_in.at[pl.ds(base, CB)], m_v)

            def edge_body(e, c2):
                for cg in range(8):
                    slf = pl.ds(cg * 16, 16)
                    sls = pl.ds(128 + cg * 16, 16)
                    f = m_v[e, slf] * par_v[0, slf] + par_v[1, slf]
                    s = m_v[e, sls] * par_v[0, sls] + par_v[1, sls]
                    sig = 1.0 / (1.0 + jnp.exp(-f))
                    t = jnp.exp(-jnp.abs(s))
                    sp = jnp.maximum(s, 0.0) + _log1p_poly(t)
                    act_v[e, slf] = sig * sp
                return c2

            lax.fori_loop(0, CB, edge_body, 0, unroll=False)
            pltpu.sync_copy(act_v, h_sh.at[didx_v], add=True)
            return carry

        lax.fori_loop(0, nchunk, chunk_body, 0, unroll=False)
        plsc.subcore_barrier()
        r0 = sid * rows_per_tile
        pltpu.sync_copy(h_sh.at[pl.ds(r0, rows_per_tile)],
                        out.at[cid, pl.ds(r0, rows_per_tile)])


def _g4_body(p_ref, node_ref, gn_ref, bn_ref, out_ref):
    h = p_ref[0] + p_ref[1]
    mean = jnp.mean(h, axis=0, keepdims=True)
    var = jnp.mean((h - mean) ** 2, axis=0, keepdims=True)
    hn = gn_ref[...] * (h - mean) * lax.rsqrt(var + 1e-5) + bn_ref[...]
    x = node_ref[...] + hn
    out_ref[...] = jnp.maximum(x, 0.0) + jnp.log1p(jnp.exp(-jnp.abs(x)))


def kernel(node_feats, edge_feats, edge_index, W_src, b_src, W_dst, b_dst,
           W_edge, b_edge, gamma_m, beta_m, gamma_n, beta_n):
    n_nodes, d = node_feats.shape
    n_edges, de = edge_feats.shape
    d2 = 2 * d
    assert n_edges % NW == 0
    epw = n_edges // NW
    assert epw % CB == 0
    nchunk = epw // CB
    n_pad = ((n_nodes + NS * 40 - 1) // (NS * 40)) * (NS * 40)

    src = edge_index[0]
    dst = edge_index[1]

    nb = 10
    bn_rows = n_nodes // nb
    h_src, h_dst = pl.pallas_call(
        _proj_body,
        grid=(nb,),
        in_specs=[
            pl.BlockSpec((bn_rows, d), lambda i: (i, 0)),
            pl.BlockSpec((d, d2), lambda i: (0, 0)),
            pl.BlockSpec((1, d2), lambda i: (0, 0)),
            pl.BlockSpec((d, d2), lambda i: (0, 0)),
            pl.BlockSpec((1, d2), lambda i: (0, 0)),
        ],
        out_specs=[
            pl.BlockSpec((bn_rows, d2), lambda i: (i, 0)),
            pl.BlockSpec((bn_rows, d2), lambda i: (i, 0)),
        ],
        out_shape=[
            jax.ShapeDtypeStruct((n_nodes, d2), jnp.float32),
            jax.ShapeDtypeStruct((n_nodes, d2), jnp.float32),
        ],
    )(node_feats, W_src, b_src.reshape(1, d2), W_dst, b_dst.reshape(1, d2))

    eb = 80
    be_rows = n_edges // eb
    eproj = pl.pallas_call(
        _eproj_body,
        grid=(eb,),
        in_specs=[
            pl.BlockSpec((be_rows, de), lambda i: (i, 0)),
            pl.BlockSpec((de, d2), lambda i: (0, 0)),
            pl.BlockSpec((1, d2), lambda i: (0, 0)),
        ],
        out_specs=pl.BlockSpec((be_rows, d2), lambda i: (i, 0)),
        out_shape=jax.ShapeDtypeStruct((n_edges, d2), jnp.float32),
    )(edge_feats, W_edge, b_edge.reshape(1, d2))

    mesh = plsc.VectorSubcoreMesh(core_axis_name="c", subcore_axis_name="s")
    s2 = functools.partial(
        pl.kernel,
        out_type=(
            jax.ShapeDtypeStruct((n_edges, d2), jnp.float32),
            jax.ShapeDtypeStruct((NW, 32, 16), jnp.float32),
        ),
        mesh=mesh,
        scratch_types=[
            pltpu.VMEM((CB,), jnp.int32),
            pltpu.VMEM((CB,), jnp.int32),
            pltpu.VMEM((CB, d2), jnp.float32),
            pltpu.VMEM((CB, d2), jnp.float32),
            pltpu.VMEM((CB, d2), jnp.float32),
            pltpu.VMEM((32, 16), jnp.float32),
            pltpu.SemaphoreType.DMA,
            pltpu.SemaphoreType.DMA,
        ],
    )(functools.partial(_s2_body, epw, nchunk))
    m_arr, stats = s2(h_src, h_dst, eproj, src, dst)

    ssum = stats.sum(axis=0)
    sum_m = ssum[:16].reshape(d2)
    sum_sq = ssum[16:].reshape(d2)
    mean = sum_m / n_edges
    var = jnp.maximum(sum_sq / n_edges - mean * mean, 0.0)
    scale = gamma_m * lax.rsqrt(var + 1e-5)
    shift = beta_m - mean * scale
    params = jnp.stack([scale, shift])

    s3 = functools.partial(
        pl.kernel,
        out_type=jax.ShapeDtypeStruct((NC, n_pad, d), jnp.float32),
        mesh=mesh,
        scratch_types=[
            pltpu.VMEM((CB,), jnp.int32),
            pltpu.VMEM((CB, d2), jnp.float32),
            pltpu.VMEM((CB, d), jnp.float32),
            pltpu.VMEM((2, d2), jnp.float32),
            pltpu.VMEM((n_pad // NS // 5, d), jnp.float32),
            pltpu.VMEM_SHARED((n_pad, d), jnp.float32),
            pltpu.SemaphoreType.DMA,
        ],
    )(functools.partial(_s3_body, n_pad, epw, nchunk))
    partials = s3(m_arr, dst, params)

    out = pl.pallas_call(
        _g4_body,
        grid=(1,),
        in_specs=[
            pl.BlockSpec((NC, n_nodes, d), lambda i: (0, 0, 0)),
            pl.BlockSpec((n_nodes, d), lambda i: (0, 0)),
            pl.BlockSpec((1, d), lambda i: (0, 0)),
            pl.BlockSpec((1, d), lambda i: (0, 0)),
        ],
        out_specs=pl.BlockSpec((n_nodes, d), lambda i: (0, 0)),
        out_shape=jax.ShapeDtypeStruct((n_nodes, d), jnp.float32),
    )(partials, node_feats, gamma_n.reshape(1, d), beta_n.reshape(1, d))
    return out


# speedup vs baseline: 1.5361x; 1.5361x over previous
import functools

import jax
import jax.numpy as jnp
from jax import lax
from jax.experimental import pallas as pl
from jax.experimental.pallas import tpu as pltpu
from jax.experimental.pallas import tpu_sc as plsc

NC = 2
NS = 16
NW = NC * NS
CB = 40

_LOG1P_C = (
    3.3869654e-08, 9.9999428e-01, -4.9983856e-01, 3.3154863e-01,
    -2.3982616e-01, 1.6582276e-01, -9.3252040e-02, 3.4849711e-02,
    -6.1514708e-03,
)


def _log1p_poly(t):
    y = jnp.full_like(t, _LOG1P_C[-1])
    for c in _LOG1P_C[-2::-1]:
        y = y * t + c
    return y


def _proj_body(node_ref, ws_ref, bs_ref, wd_ref, bd_ref, hs_ref, hd_ref):
    x = node_ref[...]
    hs_ref[...] = jnp.dot(x, ws_ref[...], preferred_element_type=jnp.float32) + bs_ref[...]
    hd_ref[...] = jnp.dot(x, wd_ref[...], preferred_element_type=jnp.float32) + bd_ref[...]


def _eproj_body(ef_ref, we_ref, be_ref, out_ref):
    out_ref[...] = (
        jnp.dot(ef_ref[...], we_ref[...], preferred_element_type=jnp.float32)
        + be_ref[...]
    )


def _s2_body(epw, nchunk, hsrc, hdst, eproj, src, dst,
             m_out, stats_out,
             si0, si1, di0, di1, a0, a1, b0, b1, c0, c1, m0, m1, acc_v,
             semg0, semg1, semw0, semw1, semi0, semi1):
    cid = lax.axis_index("c")
    sid = lax.axis_index("s")
    wid = sid * NC + cid
    SI = (si0, si1)
    DI = (di0, di1)
    A = (a0, a1)
    B = (b0, b1)
    C = (c0, c1)
    M = (m0, m1)
    SG = (semg0, semg1)
    SW = (semw0, semw1)
    SEMI = (semi0, semi1)

    for r in range(32):
        acc_v[r] = jnp.zeros((16,), jnp.float32)

    def launch_i(k, b):
        base = wid * epw + k * CB
        pltpu.async_copy(src.at[pl.ds(base, CB)], SI[b], SEMI[b])
        pltpu.async_copy(dst.at[pl.ds(base, CB)], DI[b], SEMI[b])

    def wait_i(k, b):
        base = wid * epw + k * CB
        pltpu.make_async_copy(src.at[pl.ds(base, CB)], SI[b], SEMI[b]).wait()
        pltpu.make_async_copy(dst.at[pl.ds(base, CB)], DI[b], SEMI[b]).wait()

    def launch_g(k, bn):
        base = wid * epw + k * CB
        pltpu.async_copy(hsrc.at[SI[bn]], A[bn], SG[bn])
        pltpu.async_copy(hdst.at[DI[bn]], B[bn], SG[bn])
        pltpu.async_copy(eproj.at[pl.ds(base, CB)], C[bn], SG[bn])

    def wait_g(k, bn):
        base = wid * epw + k * CB
        pltpu.make_async_copy(hsrc.at[SI[bn]], A[bn], SG[bn]).wait()
        pltpu.make_async_copy(hdst.at[DI[bn]], B[bn], SG[bn]).wait()
        pltpu.make_async_copy(eproj.at[pl.ds(base, CB)], C[bn], SG[bn]).wait()

    def launch_w(k, b):
        base = wid * epw + k * CB
        pltpu.async_copy(M[b], m_out.at[pl.ds(base, CB)], SW[b])

    def wait_w(k, b):
        base = wid * epw + k * CB
        pltpu.make_async_copy(M[b], m_out.at[pl.ds(base, CB)], SW[b]).wait()

    def compute(k, b):
        av, bv, cv, mv = A[b], B[b], C[b], M[b]

        def edge_body(e, carry):
            news = []
            newq = []
            for cg in range(16):
                sl = pl.ds(cg * 16, 16)
                m = av[e, sl] + bv[e, sl] + cv[e, sl]
                mv[e, sl] = m
                news.append(carry[cg] + m)
                newq.append(carry[16 + cg] + m * m)
            return tuple(news + newq)

        init = tuple(jnp.zeros((16,), jnp.float32) for _ in range(32))
        fin = lax.fori_loop(0, CB, edge_body, init)
        for r in range(32):
            acc_v[r] = acc_v[r] + fin[r]

    base0 = wid * epw
    pltpu.sync_copy(src.at[pl.ds(base0, CB)], SI[0])
    pltpu.sync_copy(dst.at[pl.ds(base0, CB)], DI[0])
    launch_g(0, 0)
    launch_i(1, 1)
    ng = nchunk // 2

    def body(g, carry):
        for j in range(2):
            k = 2 * g + j
            b = j
            bn = 1 - j

            def adv():
                wait_i(k + 1, bn)
                launch_g(k + 1, bn)

            if j == 0:
                adv()
            else:
                pl.when(g < ng - 1)(adv)

            @pl.when(g > 0)
            def _():
                wait_w(k - 2, b)

            wait_g(k, b)

            @pl.when(g < ng - 1)
            def _():
                launch_i(k + 2, b)

            compute(k, b)
            launch_w(k, b)
        return carry

    lax.fori_loop(0, ng, body, 0, unroll=False)
    wait_w(nchunk - 2, 0)
    wait_w(nchunk - 1, 1)
    pltpu.sync_copy(acc_v, stats_out.at[wid])


def _s3_body(n_pad, epw, nchunk, m_in, dst, params, out,
             di0, di1, di2, di3, di4, mm0, mm1,
             ac0, ac1, ac2, ac3, ac4,
             par_v, h_sh,
             sr0, sr1, ss0, ss1, ss2, ss3, ss4):
    cid = lax.axis_index("c")
    sid = lax.axis_index("s")
    wid = sid * NC + cid
    rows_per_tile = n_pad // NS
    DI = (di0, di1, di2, di3, di4)
    M = (mm0, mm1)
    ACT = (ac0, ac1, ac2, ac3, ac4)
    SR = (sr0, sr1)
    SS = (ss0, ss1, ss2, ss3, ss4)

    pltpu.sync_copy(params, par_v)

    def zrow(r, c2):
        for g in range(8):
            ac0[r, pl.ds(g * 16, 16)] = jnp.zeros((16,), jnp.float32)
        return c2

    lax.fori_loop(0, CB, zrow, 0, unroll=False)

    def zcopy(i, c2):
        pltpu.sync_copy(ac0, h_sh.at[pl.ds(sid * rows_per_tile + i * CB, CB)])
        return c2

    lax.fori_loop(0, rows_per_tile // CB, zcopy, 0, unroll=False)
    plsc.subcore_barrier()

    def launch_r(k, b2, b5):
        base = wid * epw + k * CB
        pltpu.async_copy(m_in.at[pl.ds(base, CB)], M[b2], SR[b2])
        pltpu.async_copy(dst.at[pl.ds(base, CB)], DI[b5], SR[b2])

    def wait_r(k, b2, b5):
        base = wid * epw + k * CB
        pltpu.make_async_copy(m_in.at[pl.ds(base, CB)], M[b2], SR[b2]).wait()
        pltpu.make_async_copy(dst.at[pl.ds(base, CB)], DI[b5], SR[b2]).wait()

    def launch_s(k, b5):
        pltpu.async_copy(ACT[b5], h_sh.at[DI[b5]], SS[b5], add=True)

    def wait_s(k, b5):
        pltpu.make_async_copy(ACT[b5], h_sh.at[DI[b5]], SS[b5]).wait()

    def compute(k, b2, b5):
        mv, av = M[b2], ACT[b5]

        def edge_body(e, c2):
            for cg in range(8):
                slf = pl.ds(cg * 16, 16)
                sls = pl.ds(128 + cg * 16, 16)
                f = mv[e, slf] * par_v[0, slf] + par_v[1, slf]
                s = mv[e, sls] * par_v[0, sls] + par_v[1, sls]
                sig = 1.0 / (1.0 + jnp.exp(-f))
                t = jnp.exp(-jnp.abs(s))
                sp = jnp.maximum(s, 0.0) + _log1p_poly(t)
                av[e, slf] = sig * sp
            return c2

        lax.fori_loop(0, CB, edge_body, 0, unroll=False)

    launch_r(0, 0, 0)
    ng = nchunk // 10

    def body(g, carry):
        for j in range(10):
            k = 10 * g + j
            b2 = j % 2
            b5 = j % 5
            bn2 = (j + 1) % 2
            bn5 = (j + 1) % 5

            def drain():
                wait_s(k - 4, bn5)

            if j >= 4:
                drain()
            else:
                pl.when(g > 0)(drain)

            def adv():
                launch_r(k + 1, bn2, bn5)

            if j == 9:
                pl.when(g < ng - 1)(adv)
            else:
                adv()

            wait_r(k, b2, b5)
            compute(k, b2, b5)
            launch_s(k, b5)
        return carry

    lax.fori_loop(0, ng, body, 0, unroll=False)
    for k in range(nchunk - 4, nchunk):
        wait_s(k, k % 5)
    plsc.subcore_barrier()
    r0 = sid * rows_per_tile
    pltpu.sync_copy(h_sh.at[pl.ds(r0, rows_per_tile)],
                    out.at[cid, pl.ds(r0, rows_per_tile)])


def _g4_body(p_ref, node_ref, gn_ref, bn_ref, out_ref):
    h = p_ref[0] + p_ref[1]
    mean = jnp.mean(h, axis=0, keepdims=True)
    var = jnp.mean((h - mean) ** 2, axis=0, keepdims=True)
    hn = gn_ref[...] * (h - mean) * lax.rsqrt(var + 1e-5) + bn_ref[...]
    x = node_ref[...] + hn
    out_ref[...] = jnp.maximum(x, 0.0) + jnp.log1p(jnp.exp(-jnp.abs(x)))


def kernel(node_feats, edge_feats, edge_index, W_src, b_src, W_dst, b_dst,
           W_edge, b_edge, gamma_m, beta_m, gamma_n, beta_n):
    n_nodes, d = node_feats.shape
    n_edges, de = edge_feats.shape
    d2 = 2 * d
    assert n_edges % NW == 0
    epw = n_edges // NW
    assert epw % CB == 0
    nchunk = epw // CB
    assert nchunk % 10 == 0 and nchunk >= 20
    n_pad = ((n_nodes + NS * 40 - 1) // (NS * 40)) * (NS * 40)

    src = edge_index[0]
    dst = edge_index[1]

    nb = 10
    bn_rows = n_nodes // nb
    h_src, h_dst = pl.pallas_call(
        _proj_body,
        grid=(nb,),
        in_specs=[
            pl.BlockSpec((bn_rows, d), lambda i: (i, 0)),
            pl.BlockSpec((d, d2), lambda i: (0, 0)),
            pl.BlockSpec((1, d2), lambda i: (0, 0)),
            pl.BlockSpec((d, d2), lambda i: (0, 0)),
            pl.BlockSpec((1, d2), lambda i: (0, 0)),
        ],
        out_specs=[
            pl.BlockSpec((bn_rows, d2), lambda i: (i, 0)),
            pl.BlockSpec((bn_rows, d2), lambda i: (i, 0)),
        ],
        out_shape=[
            jax.ShapeDtypeStruct((n_nodes, d2), jnp.float32),
            jax.ShapeDtypeStruct((n_nodes, d2), jnp.float32),
        ],
    )(node_feats, W_src, b_src.reshape(1, d2), W_dst, b_dst.reshape(1, d2))

    eb = 80
    be_rows = n_edges // eb
    eproj = pl.pallas_call(
        _eproj_body,
        grid=(eb,),
        in_specs=[
            pl.BlockSpec((be_rows, de), lambda i: (i, 0)),
            pl.BlockSpec((de, d2), lambda i: (0, 0)),
            pl.BlockSpec((1, d2), lambda i: (0, 0)),
        ],
        out_specs=pl.BlockSpec((be_rows, d2), lambda i: (i, 0)),
        out_shape=jax.ShapeDtypeStruct((n_edges, d2), jnp.float32),
    )(edge_feats, W_edge, b_edge.reshape(1, d2))

    mesh = plsc.VectorSubcoreMesh(core_axis_name="c", subcore_axis_name="s")
    s2 = functools.partial(
        pl.kernel,
        out_type=(
            jax.ShapeDtypeStruct((n_edges, d2), jnp.float32),
            jax.ShapeDtypeStruct((NW, 32, 16), jnp.float32),
        ),
        mesh=mesh,
        scratch_types=(
            [pltpu.VMEM((CB,), jnp.int32)] * 4
            + [pltpu.VMEM((CB, d2), jnp.float32)] * 8
            + [pltpu.VMEM((32, 16), jnp.float32)]
            + [pltpu.SemaphoreType.DMA] * 6
        ),
    )(functools.partial(_s2_body, epw, nchunk))
    m_arr, stats = s2(h_src, h_dst, eproj, src, dst)

    ssum = stats.sum(axis=0)
    sum_m = ssum[:16].reshape(d2)
    sum_sq = ssum[16:].reshape(d2)
    mean = sum_m / n_edges
    var = jnp.maximum(sum_sq / n_edges - mean * mean, 0.0)
    scale = gamma_m * lax.rsqrt(var + 1e-5)
    shift = beta_m - mean * scale
    params = jnp.stack([scale, shift])

    s3 = functools.partial(
        pl.kernel,
        out_type=jax.ShapeDtypeStruct((NC, n_pad, d), jnp.float32),
        mesh=mesh,
        scratch_types=(
            [pltpu.VMEM((CB,), jnp.int32)] * 5
            + [pltpu.VMEM((CB, d2), jnp.float32)] * 2
            + [pltpu.VMEM((CB, d), jnp.float32)] * 5
            + [pltpu.VMEM((2, d2), jnp.float32)]
            + [pltpu.VMEM_SHARED((n_pad, d), jnp.float32)]
            + [pltpu.SemaphoreType.DMA] * 7
        ),
    )(functools.partial(_s3_body, n_pad, epw, nchunk))
    partials = s3(m_arr, dst, params)

    out = pl.pallas_call(
        _g4_body,
        grid=(1,),
        in_specs=[
            pl.BlockSpec((NC, n_nodes, d), lambda i: (0, 0, 0)),
            pl.BlockSpec((n_nodes, d), lambda i: (0, 0)),
            pl.BlockSpec((1, d), lambda i: (0, 0)),
            pl.BlockSpec((1, d), lambda i: (0, 0)),
        ],
        out_specs=pl.BlockSpec((n_nodes, d), lambda i: (0, 0)),
        out_shape=jax.ShapeDtypeStruct((n_nodes, d), jnp.float32),
    )(partials, node_feats, gamma_n.reshape(1, d), beta_n.reshape(1, d))
    return out
